# Initial kernel scaffold; baseline (speedup 1.0000x reference)
import jax
import jax.numpy as jnp
from jax.experimental import pallas as pl


def kernel(x, edge_attr, edge_index, en1_w1, en1_b1, en1_w2, en1_b2, en2_w1, en2_b1, en2_w2, en2_b2, root1, bias1, root2, bias2, q_w, q_b):
    raise NotImplementedError("write your pallas kernel here")


# speedup vs baseline: 3.3261x; 3.3261x over previous
import functools

import jax
import jax.numpy as jnp
from jax import lax
from jax.experimental import pallas as pl
from jax.experimental.pallas import tpu as pltpu
from jax.experimental.pallas import tpu_sc as plsc

N = 10000
E = 160000
DIN = 128
DE = 16
H = 16
HH = H * H
PW = HH + H
AW = 32
NPAD = 10240

NC = 2
NS = 16
NW = NC * NS
EPT = E // NW
CHUNK = 125
NCH = EPT // CHUNK
NPT = NPAD // NS


def _z_body(ea_ref, w1a_ref, b1a_ref, w1b_ref, b1b_ref, z1_ref, z2_ref):
    ea = ea_ref[...]
    z1_ref[...] = jnp.maximum(
        jnp.dot(ea, w1a_ref[...], preferred_element_type=jnp.float32)
        + b1a_ref[...], 0.0)
    z2_ref[...] = jnp.maximum(
        jnp.dot(ea, w1b_ref[...], preferred_element_type=jnp.float32)
        + b1b_ref[...], 0.0)


def _p_body(x_ref, w_ref, p_ref):
    p_ref[...] = jnp.dot(x_ref[...], w_ref[...],
                         preferred_element_type=jnp.float32)


def _mid_body(acc_ref, x_ref, root_ref, bias_ref, wcat_ref, h_ref, p_ref):
    s = acc_ref[0] + acc_ref[1]
    agg = s[:, 0:H] / jnp.maximum(s[:, H:H + 1], 1.0)
    h = jnp.maximum(
        agg + jnp.dot(x_ref[...], root_ref[...],
                      preferred_element_type=jnp.float32) + bias_ref[...], 0.0)
    h_ref[...] = h
    p_ref[...] = jnp.dot(h, wcat_ref[...], preferred_element_type=jnp.float32)


def _fin_body(acc_ref, h_ref, root_ref, bias_ref, qw_ref, qb_ref, out_ref):
    s = acc_ref[0] + acc_ref[1]
    agg = s[:, 0:H] / jnp.maximum(s[:, H:H + 1], 1.0)
    h2 = jnp.maximum(
        agg + jnp.dot(h_ref[...], root_ref[...],
                      preferred_element_type=jnp.float32) + bias_ref[...], 0.0)
    out_ref[...] = jnp.dot(h2, qw_ref[...],
                           preferred_element_type=jnp.float32) + qb_ref[0, 0]


def _sc_msg_body(p_hbm, z_hbm, src_hbm, dst_hbm, out_hbm,
                 idx_s, idx_d, z_v, rows_v, outbuf_v, zero_v, acc_sh, sem):
    cid = lax.axis_index("c")
    sid = lax.axis_index("s")
    wid = cid * NS + sid

    zvec = jnp.zeros((16,), jnp.float32)

    def zero_row(i, carry):
        zero_v[i, pl.ds(0, 16)] = zvec
        zero_v[i, pl.ds(16, 16)] = zvec
        return carry
    lax.fori_loop(0, NPT, zero_row, 0)
    pltpu.sync_copy(zero_v, acc_sh.at[pl.ds(sid * NPT, NPT)])

    pat = jnp.where(lax.iota(jnp.int32, 16) == 0, 1.0, 0.0).astype(jnp.float32)

    def init_row(i, carry):
        outbuf_v[i, pl.ds(H, 16)] = pat
        return carry
    lax.fori_loop(0, CHUNK, init_row, 0)

    plsc.subcore_barrier()

    def chunk_body(c, carry):
        pltpu.sync_copy(src_hbm.at[wid, c], idx_s)
        pltpu.sync_copy(dst_hbm.at[wid, c], idx_d)
        pltpu.async_copy(p_hbm.at[idx_s], rows_v, sem).wait()
        pltpu.sync_copy(z_hbm.at[wid, c], z_v)

        def edge_body(i, ecarry):
            zrow = z_v[i, pl.ds(0, H)]
            m = rows_v[i, pl.ds(HH, 16)]
            for k in range(H):
                m = m + zrow[k] * rows_v[i, pl.ds(k * H, 16)]
            outbuf_v[i, pl.ds(0, 16)] = m
            return ecarry
        lax.fori_loop(0, CHUNK, edge_body, 0)

        pltpu.sync_copy(outbuf_v, acc_sh.at[idx_d], add=True)
        return carry
    lax.fori_loop(0, NCH, chunk_body, 0)

    plsc.subcore_barrier()
    pltpu.sync_copy(acc_sh.at[pl.ds(sid * NPT, NPT)],
                    out_hbm.at[cid, pl.ds(sid * NPT, NPT)])


_sc_msg = pl.kernel(
    _sc_msg_body,
    out_type=jax.ShapeDtypeStruct((NC, NPAD, AW), jnp.float32),
    mesh=plsc.VectorSubcoreMesh(core_axis_name="c", subcore_axis_name="s"),
    compiler_params=pltpu.CompilerParams(use_tc_tiling_on_sc=False),
    scratch_types=[
        pltpu.VMEM((CHUNK,), jnp.int32),
        pltpu.VMEM((CHUNK,), jnp.int32),
        pltpu.VMEM((CHUNK, H), jnp.float32),
        pltpu.VMEM((CHUNK, PW), jnp.float32),
        pltpu.VMEM((CHUNK, AW), jnp.float32),
        pltpu.VMEM((NPT, AW), jnp.float32),
        pltpu.VMEM_SHARED((NPAD, AW), jnp.float32),
        pltpu.SemaphoreType.DMA,
    ],
)


def _make_wcat(w2, b2, din):
    w2p = w2.reshape(H, din, H).transpose(1, 0, 2).reshape(din, HH)
    b2r = b2.reshape(din, H)
    return jnp.concatenate([w2p, b2r], axis=1)


@jax.jit
def kernel(x, edge_attr, edge_index, en1_w1, en1_b1, en1_w2, en1_b2,
           en2_w1, en2_b1, en2_w2, en2_b2, root1, bias1, root2, bias2,
           q_w, q_b):
    src = edge_index[0].reshape(NW, NCH, CHUNK)
    dst = edge_index[1].reshape(NW, NCH, CHUNK)

    wcat1 = _make_wcat(en1_w2, en1_b2, DIN)
    wcat2 = _make_wcat(en2_w2, en2_b2, H)

    eb = 8000
    z1, z2 = pl.pallas_call(
        _z_body,
        grid=(E // eb,),
        in_specs=[
            pl.BlockSpec((eb, DE), lambda i: (i, 0)),
            pl.BlockSpec((DE, H), lambda i: (0, 0)),
            pl.BlockSpec((1, H), lambda i: (0, 0)),
            pl.BlockSpec((DE, H), lambda i: (0, 0)),
            pl.BlockSpec((1, H), lambda i: (0, 0)),
        ],
        out_specs=[
            pl.BlockSpec((eb, H), lambda i: (i, 0)),
            pl.BlockSpec((eb, H), lambda i: (i, 0)),
        ],
        out_shape=[
            jax.ShapeDtypeStruct((E, H), jnp.float32),
            jax.ShapeDtypeStruct((E, H), jnp.float32),
        ],
    )(edge_attr, en1_w1, en1_b1.reshape(1, H), en2_w1, en2_b1.reshape(1, H))

    z1g = z1.reshape(NW, NCH, CHUNK, H)
    z2g = z2.reshape(NW, NCH, CHUNK, H)

    nb = 2000
    p1 = pl.pallas_call(
        _p_body,
        grid=(N // nb,),
        in_specs=[
            pl.BlockSpec((nb, DIN), lambda i: (i, 0)),
            pl.BlockSpec((DIN, PW), lambda i: (0, 0)),
        ],
        out_specs=pl.BlockSpec((nb, PW), lambda i: (i, 0)),
        out_shape=jax.ShapeDtypeStruct((N, PW), jnp.float32),
    )(x, wcat1)

    acc1 = _sc_msg(p1, z1g, src, dst)[:, :N, :]

    h, p2 = pl.pallas_call(
        _mid_body,
        grid=(N // nb,),
        in_specs=[
            pl.BlockSpec((NC, nb, AW), lambda i: (0, i, 0)),
            pl.BlockSpec((nb, DIN), lambda i: (i, 0)),
            pl.BlockSpec((DIN, H), lambda i: (0, 0)),
            pl.BlockSpec((1, H), lambda i: (0, 0)),
            pl.BlockSpec((H, PW), lambda i: (0, 0)),
        ],
        out_specs=[
            pl.BlockSpec((nb, H), lambda i: (i, 0)),
            pl.BlockSpec((nb, PW), lambda i: (i, 0)),
        ],
        out_shape=[
            jax.ShapeDtypeStruct((N, H), jnp.float32),
            jax.ShapeDtypeStruct((N, PW), jnp.float32),
        ],
    )(acc1, x, root1, bias1.reshape(1, H), wcat2)

    acc2 = _sc_msg(p2, z2g, src, dst)[:, :N, :]

    out2d = pl.pallas_call(
        _fin_body,
        grid=(N // nb,),
        in_specs=[
            pl.BlockSpec((NC, nb, AW), lambda i: (0, i, 0)),
            pl.BlockSpec((nb, H), lambda i: (i, 0)),
            pl.BlockSpec((H, H), lambda i: (0, 0)),
            pl.BlockSpec((1, H), lambda i: (0, 0)),
            pl.BlockSpec((H, 1), lambda i: (0, 0)),
            pl.BlockSpec((1, 1), lambda i: (0, 0)),
        ],
        out_specs=pl.BlockSpec((nb, 1), lambda i: (i, 0)),
        out_shape=jax.ShapeDtypeStruct((N, 1), jnp.float32),
    )(acc2, h, root2, bias2.reshape(1, H), q_w, q_b.reshape(1, 1))

    return out2d[:, 0]


# speedup vs baseline: 3.9778x; 1.1959x over previous
import functools

import jax
import jax.numpy as jnp
from jax import lax
from jax.experimental import pallas as pl
from jax.experimental.pallas import tpu as pltpu
from jax.experimental.pallas import tpu_sc as plsc

N = 10000
E = 160000
DIN = 128
DE = 16
H = 16
HH = H * H
PW = HH + H
AW = 32
NPAD = 10240

NC = 2
NS = 16
NW = NC * NS
EPT = E // NW
CHUNK = 125
NCH = EPT // CHUNK
NPT = NPAD // NS


def _z_body(ea_ref, w1a_ref, b1a_ref, w1b_ref, b1b_ref, z1_ref, z2_ref):
    ea = ea_ref[...]
    z1_ref[...] = jnp.maximum(
        jnp.dot(ea, w1a_ref[...], preferred_element_type=jnp.float32)
        + b1a_ref[...], 0.0)
    z2_ref[...] = jnp.maximum(
        jnp.dot(ea, w1b_ref[...], preferred_element_type=jnp.float32)
        + b1b_ref[...], 0.0)


def _p_body(x_ref, w_ref, p_ref):
    p_ref[...] = jnp.dot(x_ref[...], w_ref[...],
                         preferred_element_type=jnp.float32)


def _mid_body(acc_ref, x_ref, root_ref, bias_ref, wcat_ref, h_ref, p_ref):
    s = acc_ref[0] + acc_ref[1]
    agg = s[:, 0:H] / jnp.maximum(s[:, H:H + 1], 1.0)
    h = jnp.maximum(
        agg + jnp.dot(x_ref[...], root_ref[...],
                      preferred_element_type=jnp.float32) + bias_ref[...], 0.0)
    h_ref[...] = h
    p_ref[...] = jnp.dot(h, wcat_ref[...], preferred_element_type=jnp.float32)


def _fin_body(acc_ref, h_ref, root_ref, bias_ref, qw_ref, qb_ref, out_ref):
    s = acc_ref[0] + acc_ref[1]
    agg = s[:, 0:H] / jnp.maximum(s[:, H:H + 1], 1.0)
    h2 = jnp.maximum(
        agg + jnp.dot(h_ref[...], root_ref[...],
                      preferred_element_type=jnp.float32) + bias_ref[...], 0.0)
    out_ref[...] = jnp.dot(h2, qw_ref[...],
                           preferred_element_type=jnp.float32) + qb_ref[0, 0]


def _sc_msg_body(p_hbm, z_hbm, src_hbm, dst_hbm, out_hbm,
                 idx_s0, idx_s1, idx_d, z_v, rows_v0, rows_v1, outbuf_v,
                 zero_v, acc_sh, sem0, sem1):
    cid = lax.axis_index("c")
    sid = lax.axis_index("s")
    wid = cid * NS + sid

    idx_bufs = (idx_s0, idx_s1)
    row_bufs = (rows_v0, rows_v1)
    sems = (sem0, sem1)

    zvec = jnp.zeros((16,), jnp.float32)

    def zero_row(i, carry):
        zero_v[i, pl.ds(0, 16)] = zvec
        zero_v[i, pl.ds(16, 16)] = zvec
        return carry
    lax.fori_loop(0, NPT, zero_row, 0)
    pltpu.sync_copy(zero_v, acc_sh.at[pl.ds(sid * NPT, NPT)])

    pat = jnp.where(lax.iota(jnp.int32, 16) == 0, 1.0, 0.0).astype(jnp.float32)

    def init_row(i, carry):
        outbuf_v[i, pl.ds(H, 16)] = pat
        return carry
    lax.fori_loop(0, CHUNK, init_row, 0)

    plsc.subcore_barrier()

    pltpu.sync_copy(src_hbm.at[wid, 0], idx_s0)
    pltpu.async_copy(p_hbm.at[idx_s0], rows_v0, sem0)

    def process(c, buf):
        rows_v = row_bufs[buf]
        @pl.when(c + 1 < NCH)
        def _():
            pltpu.sync_copy(src_hbm.at[wid, c + 1], idx_bufs[1 - buf])
            pltpu.async_copy(p_hbm.at[idx_bufs[1 - buf]],
                             row_bufs[1 - buf], sems[1 - buf])
        pltpu.make_async_copy(p_hbm.at[idx_bufs[buf]], rows_v,
                              sems[buf]).wait()
        pltpu.sync_copy(z_hbm.at[wid, c], z_v)
        pltpu.sync_copy(dst_hbm.at[wid, c], idx_d)

        def edge_body(i, ecarry):
            zrow = z_v[i, pl.ds(0, H)]
            m = rows_v[i, pl.ds(HH, 16)]
            for k in range(H):
                m = m + zrow[k] * rows_v[i, pl.ds(k * H, 16)]
            outbuf_v[i, pl.ds(0, 16)] = m
            return ecarry
        lax.fori_loop(0, CHUNK, edge_body, 0)

        pltpu.sync_copy(outbuf_v, acc_sh.at[idx_d], add=True)

    def chunk_pair(g, carry):
        process(2 * g, 0)
        process(2 * g + 1, 1)
        return carry
    lax.fori_loop(0, NCH // 2, chunk_pair, 0)

    plsc.subcore_barrier()
    pltpu.sync_copy(acc_sh.at[pl.ds(sid * NPT, NPT)],
                    out_hbm.at[cid, pl.ds(sid * NPT, NPT)])


_sc_msg = pl.kernel(
    _sc_msg_body,
    out_type=jax.ShapeDtypeStruct((NC, NPAD, AW), jnp.float32),
    mesh=plsc.VectorSubcoreMesh(core_axis_name="c", subcore_axis_name="s"),
    compiler_params=pltpu.CompilerParams(use_tc_tiling_on_sc=False),
    scratch_types=[
        pltpu.VMEM((CHUNK,), jnp.int32),
        pltpu.VMEM((CHUNK,), jnp.int32),
        pltpu.VMEM((CHUNK,), jnp.int32),
        pltpu.VMEM((CHUNK, H), jnp.float32),
        pltpu.VMEM((CHUNK, PW), jnp.float32),
        pltpu.VMEM((CHUNK, PW), jnp.float32),
        pltpu.VMEM((CHUNK, AW), jnp.float32),
        pltpu.VMEM((NPT, AW), jnp.float32),
        pltpu.VMEM_SHARED((NPAD, AW), jnp.float32),
        pltpu.SemaphoreType.DMA,
        pltpu.SemaphoreType.DMA,
    ],
)


def _make_wcat(w2, b2, din):
    w2p = w2.reshape(H, din, H).transpose(1, 0, 2).reshape(din, HH)
    b2r = b2.reshape(din, H)
    return jnp.concatenate([w2p, b2r], axis=1)


@jax.jit
def kernel(x, edge_attr, edge_index, en1_w1, en1_b1, en1_w2, en1_b2,
           en2_w1, en2_b1, en2_w2, en2_b2, root1, bias1, root2, bias2,
           q_w, q_b):
    src = edge_index[0].reshape(NW, NCH, CHUNK)
    dst = edge_index[1].reshape(NW, NCH, CHUNK)

    wcat1 = _make_wcat(en1_w2, en1_b2, DIN)
    wcat2 = _make_wcat(en2_w2, en2_b2, H)

    eb = 8000
    z1, z2 = pl.pallas_call(
        _z_body,
        grid=(E // eb,),
        in_specs=[
            pl.BlockSpec((eb, DE), lambda i: (i, 0)),
            pl.BlockSpec((DE, H), lambda i: (0, 0)),
            pl.BlockSpec((1, H), lambda i: (0, 0)),
            pl.BlockSpec((DE, H), lambda i: (0, 0)),
            pl.BlockSpec((1, H), lambda i: (0, 0)),
        ],
        out_specs=[
            pl.BlockSpec((eb, H), lambda i: (i, 0)),
            pl.BlockSpec((eb, H), lambda i: (i, 0)),
        ],
        out_shape=[
            jax.ShapeDtypeStruct((E, H), jnp.float32),
            jax.ShapeDtypeStruct((E, H), jnp.float32),
        ],
    )(edge_attr, en1_w1, en1_b1.reshape(1, H), en2_w1, en2_b1.reshape(1, H))

    z1g = z1.reshape(NW, NCH, CHUNK, H)
    z2g = z2.reshape(NW, NCH, CHUNK, H)

    nb = 2000
    p1 = pl.pallas_call(
        _p_body,
        grid=(N // nb,),
        in_specs=[
            pl.BlockSpec((nb, DIN), lambda i: (i, 0)),
            pl.BlockSpec((DIN, PW), lambda i: (0, 0)),
        ],
        out_specs=pl.BlockSpec((nb, PW), lambda i: (i, 0)),
        out_shape=jax.ShapeDtypeStruct((N, PW), jnp.float32),
    )(x, wcat1)

    acc1 = _sc_msg(p1, z1g, src, dst)[:, :N, :]

    h, p2 = pl.pallas_call(
        _mid_body,
        grid=(N // nb,),
        in_specs=[
            pl.BlockSpec((NC, nb, AW), lambda i: (0, i, 0)),
            pl.BlockSpec((nb, DIN), lambda i: (i, 0)),
            pl.BlockSpec((DIN, H), lambda i: (0, 0)),
            pl.BlockSpec((1, H), lambda i: (0, 0)),
            pl.BlockSpec((H, PW), lambda i: (0, 0)),
        ],
        out_specs=[
            pl.BlockSpec((nb, H), lambda i: (i, 0)),
            pl.BlockSpec((nb, PW), lambda i: (i, 0)),
        ],
        out_shape=[
            jax.ShapeDtypeStruct((N, H), jnp.float32),
            jax.ShapeDtypeStruct((N, PW), jnp.float32),
        ],
    )(acc1, x, root1, bias1.reshape(1, H), wcat2)

    acc2 = _sc_msg(p2, z2g, src, dst)[:, :N, :]

    out2d = pl.pallas_call(
        _fin_body,
        grid=(N // nb,),
        in_specs=[
            pl.BlockSpec((NC, nb, AW), lambda i: (0, i, 0)),
            pl.BlockSpec((nb, H), lambda i: (i, 0)),
            pl.BlockSpec((H, H), lambda i: (0, 0)),
            pl.BlockSpec((1, H), lambda i: (0, 0)),
            pl.BlockSpec((H, 1), lambda i: (0, 0)),
            pl.BlockSpec((1, 1), lambda i: (0, 0)),
        ],
        out_specs=pl.BlockSpec((nb, 1), lambda i: (i, 0)),
        out_shape=jax.ShapeDtypeStruct((N, 1), jnp.float32),
    )(acc2, h, root2, bias2.reshape(1, H), q_w, q_b.reshape(1, 1))

    return out2d[:, 0]


# speedup vs baseline: 4.8381x; 1.2163x over previous
import functools

import jax
import jax.numpy as jnp
from jax import lax
from jax.experimental import pallas as pl
from jax.experimental.pallas import tpu as pltpu
from jax.experimental.pallas import tpu_sc as plsc

N = 10000
E = 160000
DIN = 128
DE = 16
H = 16
HH = H * H
PW = HH + H
AW = 32
NPAD = 10240

NC = 2
NS = 16
NW = NC * NS
EPT = E // NW
CHUNK = 125
NCH = EPT // CHUNK
NPT = NPAD // NS


def _z_body(ea_ref, w1a_ref, b1a_ref, w1b_ref, b1b_ref, z1_ref, z2_ref):
    ea = ea_ref[...]
    z1_ref[...] = jnp.maximum(
        jnp.dot(ea, w1a_ref[...], preferred_element_type=jnp.float32)
        + b1a_ref[...], 0.0)
    z2_ref[...] = jnp.maximum(
        jnp.dot(ea, w1b_ref[...], preferred_element_type=jnp.float32)
        + b1b_ref[...], 0.0)


def _p_body(x_ref, w_ref, p_ref):
    p_ref[...] = jnp.dot(x_ref[...], w_ref[...],
                         preferred_element_type=jnp.float32)


def _mid_body(acc_ref, x_ref, root_ref, bias_ref, wcat_ref, h_ref, p_ref):
    s = acc_ref[0] + acc_ref[1]
    agg = s[:, 0:H] / jnp.maximum(s[:, H:H + 1], 1.0)
    h = jnp.maximum(
        agg + jnp.dot(x_ref[...], root_ref[...],
                      preferred_element_type=jnp.float32) + bias_ref[...], 0.0)
    h_ref[...] = h
    p_ref[...] = jnp.dot(h, wcat_ref[...], preferred_element_type=jnp.float32)


def _fin_body(acc_ref, h_ref, root_ref, bias_ref, qw_ref, qb_ref, out_ref):
    s = acc_ref[0] + acc_ref[1]
    agg = s[:, 0:H] / jnp.maximum(s[:, H:H + 1], 1.0)
    h2 = jnp.maximum(
        agg + jnp.dot(h_ref[...], root_ref[...],
                      preferred_element_type=jnp.float32) + bias_ref[...], 0.0)
    out_ref[...] = jnp.dot(h2, qw_ref[...],
                           preferred_element_type=jnp.float32) + qb_ref[0, 0]


def _sc_msg_body(p_hbm, z_hbm, src_hbm, dst_hbm, out_hbm,
                 idx_s0, idx_s1, idx_d, z_v, rows_v0, rows_v1, outbuf_v,
                 zero_v, acc_sh, sem0, sem1):
    cid = lax.axis_index("c")
    sid = lax.axis_index("s")
    wid = cid * NS + sid

    idx_bufs = (idx_s0, idx_s1)
    row_bufs = (rows_v0, rows_v1)
    sems = (sem0, sem1)

    zvec = jnp.zeros((16,), jnp.float32)

    def zero_row(i, carry):
        zero_v[i, pl.ds(0, 16)] = zvec
        zero_v[i, pl.ds(16, 16)] = zvec
        return carry
    lax.fori_loop(0, NPT, zero_row, 0)
    pltpu.sync_copy(zero_v, acc_sh.at[pl.ds(sid * NPT, NPT)])

    pat = jnp.where(lax.iota(jnp.int32, 16) == 0, 1.0, 0.0).astype(jnp.float32)

    def init_row(i, carry):
        outbuf_v[i, pl.ds(H, 16)] = pat
        return carry
    lax.fori_loop(0, CHUNK, init_row, 0)

    plsc.subcore_barrier()

    pltpu.sync_copy(src_hbm.at[wid, 0], idx_s0)
    pltpu.async_copy(p_hbm.at[idx_s0], rows_v0, sem0)

    def process(c, buf):
        rows_v = row_bufs[buf]
        @pl.when(c + 1 < NCH)
        def _():
            pltpu.sync_copy(src_hbm.at[wid, c + 1], idx_bufs[1 - buf])
            pltpu.async_copy(p_hbm.at[idx_bufs[1 - buf]],
                             row_bufs[1 - buf], sems[1 - buf])
        pltpu.make_async_copy(p_hbm.at[idx_bufs[buf]], rows_v,
                              sems[buf]).wait()
        pltpu.sync_copy(z_hbm.at[wid, c], z_v)
        pltpu.sync_copy(dst_hbm.at[wid, c], idx_d)

        @plsc.parallel_loop(0, CHUNK, unroll=5)
        def _(i):
            zrow = z_v[i, pl.ds(0, H)]
            m = rows_v[i, pl.ds(HH, 16)]
            for k in range(H):
                m = m + zrow[k] * rows_v[i, pl.ds(k * H, 16)]
            outbuf_v[i, pl.ds(0, 16)] = m

        pltpu.sync_copy(outbuf_v, acc_sh.at[idx_d], add=True)

    def chunk_pair(g, carry):
        process(2 * g, 0)
        process(2 * g + 1, 1)
        return carry
    lax.fori_loop(0, NCH // 2, chunk_pair, 0)

    plsc.subcore_barrier()
    pltpu.sync_copy(acc_sh.at[pl.ds(sid * NPT, NPT)],
                    out_hbm.at[cid, pl.ds(sid * NPT, NPT)])


_sc_msg = pl.kernel(
    _sc_msg_body,
    out_type=jax.ShapeDtypeStruct((NC, NPAD, AW), jnp.float32),
    mesh=plsc.VectorSubcoreMesh(core_axis_name="c", subcore_axis_name="s"),
    compiler_params=pltpu.CompilerParams(use_tc_tiling_on_sc=False),
    scratch_types=[
        pltpu.VMEM((CHUNK,), jnp.int32),
        pltpu.VMEM((CHUNK,), jnp.int32),
        pltpu.VMEM((CHUNK,), jnp.int32),
        pltpu.VMEM((CHUNK, H), jnp.float32),
        pltpu.VMEM((CHUNK, PW), jnp.float32),
        pltpu.VMEM((CHUNK, PW), jnp.float32),
        pltpu.VMEM((CHUNK, AW), jnp.float32),
        pltpu.VMEM((NPT, AW), jnp.float32),
        pltpu.VMEM_SHARED((NPAD, AW), jnp.float32),
        pltpu.SemaphoreType.DMA,
        pltpu.SemaphoreType.DMA,
    ],
)


def _make_wcat(w2, b2, din):
    w2p = w2.reshape(H, din, H).transpose(1, 0, 2).reshape(din, HH)
    b2r = b2.reshape(din, H)
    return jnp.concatenate([w2p, b2r], axis=1)


@jax.jit
def kernel(x, edge_attr, edge_index, en1_w1, en1_b1, en1_w2, en1_b2,
           en2_w1, en2_b1, en2_w2, en2_b2, root1, bias1, root2, bias2,
           q_w, q_b):
    src = edge_index[0].reshape(NW, NCH, CHUNK)
    dst = edge_index[1].reshape(NW, NCH, CHUNK)

    wcat1 = _make_wcat(en1_w2, en1_b2, DIN)
    wcat2 = _make_wcat(en2_w2, en2_b2, H)

    eb = 8000
    z1, z2 = pl.pallas_call(
        _z_body,
        grid=(E // eb,),
        in_specs=[
            pl.BlockSpec((eb, DE), lambda i: (i, 0)),
            pl.BlockSpec((DE, H), lambda i: (0, 0)),
            pl.BlockSpec((1, H), lambda i: (0, 0)),
            pl.BlockSpec((DE, H), lambda i: (0, 0)),
            pl.BlockSpec((1, H), lambda i: (0, 0)),
        ],
        out_specs=[
            pl.BlockSpec((eb, H), lambda i: (i, 0)),
            pl.BlockSpec((eb, H), lambda i: (i, 0)),
        ],
        out_shape=[
            jax.ShapeDtypeStruct((E, H), jnp.float32),
            jax.ShapeDtypeStruct((E, H), jnp.float32),
        ],
    )(edge_attr, en1_w1, en1_b1.reshape(1, H), en2_w1, en2_b1.reshape(1, H))

    z1g = z1.reshape(NW, NCH, CHUNK, H)
    z2g = z2.reshape(NW, NCH, CHUNK, H)

    nb = 2000
    p1 = pl.pallas_call(
        _p_body,
        grid=(N // nb,),
        in_specs=[
            pl.BlockSpec((nb, DIN), lambda i: (i, 0)),
            pl.BlockSpec((DIN, PW), lambda i: (0, 0)),
        ],
        out_specs=pl.BlockSpec((nb, PW), lambda i: (i, 0)),
        out_shape=jax.ShapeDtypeStruct((N, PW), jnp.float32),
    )(x, wcat1)

    acc1 = _sc_msg(p1, z1g, src, dst)[:, :N, :]

    h, p2 = pl.pallas_call(
        _mid_body,
        grid=(N // nb,),
        in_specs=[
            pl.BlockSpec((NC, nb, AW), lambda i: (0, i, 0)),
            pl.BlockSpec((nb, DIN), lambda i: (i, 0)),
            pl.BlockSpec((DIN, H), lambda i: (0, 0)),
            pl.BlockSpec((1, H), lambda i: (0, 0)),
            pl.BlockSpec((H, PW), lambda i: (0, 0)),
        ],
        out_specs=[
            pl.BlockSpec((nb, H), lambda i: (i, 0)),
            pl.BlockSpec((nb, PW), lambda i: (i, 0)),
        ],
        out_shape=[
            jax.ShapeDtypeStruct((N, H), jnp.float32),
            jax.ShapeDtypeStruct((N, PW), jnp.float32),
        ],
    )(acc1, x, root1, bias1.reshape(1, H), wcat2)

    acc2 = _sc_msg(p2, z2g, src, dst)[:, :N, :]

    out2d = pl.pallas_call(
        _fin_body,
        grid=(N // nb,),
        in_specs=[
            pl.BlockSpec((NC, nb, AW), lambda i: (0, i, 0)),
            pl.BlockSpec((nb, H), lambda i: (i, 0)),
            pl.BlockSpec((H, H), lambda i: (0, 0)),
            pl.BlockSpec((1, H), lambda i: (0, 0)),
            pl.BlockSpec((H, 1), lambda i: (0, 0)),
            pl.BlockSpec((1, 1), lambda i: (0, 0)),
        ],
        out_specs=pl.BlockSpec((nb, 1), lambda i: (i, 0)),
        out_shape=jax.ShapeDtypeStruct((N, 1), jnp.float32),
    )(acc2, h, root2, bias2.reshape(1, H), q_w, q_b.reshape(1, 1))

    return out2d[:, 0]


# speedup vs baseline: 5.9663x; 1.2332x over previous
import functools

import jax
import jax.numpy as jnp
from jax import lax
from jax.experimental import pallas as pl
from jax.experimental.pallas import tpu as pltpu
from jax.experimental.pallas import tpu_sc as plsc

N = 10000
E = 160000
DIN = 128
DE = 16
H = 16
HH = H * H
PW = HH + H
AW = 32
NPAD = 10240

NC = 2
NS = 16
NW = NC * NS
EPT = E // NW
CHUNK = 125
NCH = EPT // CHUNK
NPT = NPAD // NS


def _pre_body(ea_ref, w1a_ref, b1a_ref, w1b_ref, b1b_ref, x_ref, wcat_ref,
              z1_ref, z2_ref, p_ref):
    ea = ea_ref[...]
    z1_ref[...] = jnp.maximum(
        jnp.dot(ea, w1a_ref[...], preferred_element_type=jnp.float32)
        + b1a_ref[...], 0.0)
    z2_ref[...] = jnp.maximum(
        jnp.dot(ea, w1b_ref[...], preferred_element_type=jnp.float32)
        + b1b_ref[...], 0.0)
    p_ref[...] = jnp.dot(x_ref[...], wcat_ref[...],
                         preferred_element_type=jnp.float32)


def _mid_body(acc_ref, x_ref, root_ref, bias_ref, wcat_ref, h_ref, p_ref):
    s = acc_ref[0] + acc_ref[1]
    agg = s[:, 0:H] / jnp.maximum(s[:, H:H + 1], 1.0)
    h = jnp.maximum(
        agg + jnp.dot(x_ref[...], root_ref[...],
                      preferred_element_type=jnp.float32) + bias_ref[...], 0.0)
    h_ref[...] = h
    p_ref[...] = jnp.dot(h, wcat_ref[...], preferred_element_type=jnp.float32)


def _fin_body(acc_ref, h_ref, root_ref, bias_ref, qw_ref, qb_ref, out_ref):
    s = acc_ref[0] + acc_ref[1]
    agg = s[:, 0:H] / jnp.maximum(s[:, H:H + 1], 1.0)
    h2 = jnp.maximum(
        agg + jnp.dot(h_ref[...], root_ref[...],
                      preferred_element_type=jnp.float32) + bias_ref[...], 0.0)
    out_ref[...] = jnp.dot(h2, qw_ref[...],
                           preferred_element_type=jnp.float32) + qb_ref[0, 0]


def _sc_msg_body(p_hbm, z_hbm, src_hbm, dst_hbm, out_hbm,
                 idx_s0, idx_s1, idx_d0, idx_d1, z_v0, z_v1,
                 rows_v0, rows_v1, outbuf_v0, outbuf_v1, zero_v, acc_sh,
                 sem_r0, sem_r1, sem_z0, sem_z1, sem_d0, sem_d1):
    cid = lax.axis_index("c")
    sid = lax.axis_index("s")
    wid = cid * NS + sid

    idx_s = (idx_s0, idx_s1)
    idx_d = (idx_d0, idx_d1)
    z_bufs = (z_v0, z_v1)
    row_bufs = (rows_v0, rows_v1)
    out_bufs = (outbuf_v0, outbuf_v1)
    sem_r = (sem_r0, sem_r1)
    sem_z = (sem_z0, sem_z1)
    sem_d = (sem_d0, sem_d1)

    zvec = jnp.zeros((16,), jnp.float32)

    @plsc.parallel_loop(0, NPT, unroll=8)
    def _(i):
        zero_v[i, pl.ds(0, 16)] = zvec
        zero_v[i, pl.ds(16, 16)] = zvec
    pltpu.sync_copy(zero_v, acc_sh.at[pl.ds(sid * NPT, NPT)])

    pat = jnp.where(lax.iota(jnp.int32, 16) == 0, 1.0, 0.0).astype(jnp.float32)

    @plsc.parallel_loop(0, CHUNK, unroll=5)
    def _(i):
        outbuf_v0[i, pl.ds(H, 16)] = pat
        outbuf_v1[i, pl.ds(H, 16)] = pat

    plsc.subcore_barrier()

    def issue(c, buf):
        pltpu.sync_copy(src_hbm.at[wid, c], idx_s[buf])
        pltpu.async_copy(p_hbm.at[idx_s[buf]], row_bufs[buf], sem_r[buf])
        pltpu.async_copy(z_hbm.at[wid, c], z_bufs[buf], sem_z[buf])
        pltpu.async_copy(dst_hbm.at[wid, c], idx_d[buf], sem_d[buf])

    issue(0, 0)

    def process(c, buf):
        rows_v = row_bufs[buf]
        z_v = z_bufs[buf]
        outbuf_v = out_bufs[buf]

        @pl.when(c + 1 < NCH)
        def _():
            issue(c + 1, 1 - buf)

        pltpu.make_async_copy(p_hbm.at[idx_s[buf]], rows_v,
                              sem_r[buf]).wait()
        pltpu.make_async_copy(z_hbm.at[wid, c], z_v, sem_z[buf]).wait()
        pltpu.make_async_copy(dst_hbm.at[wid, c], idx_d[buf],
                              sem_d[buf]).wait()

        @plsc.parallel_loop(0, CHUNK, unroll=5)
        def _(i):
            zrow = z_v[i, pl.ds(0, H)]
            m = rows_v[i, pl.ds(HH, 16)]
            for k in range(H):
                m = m + zrow[k] * rows_v[i, pl.ds(k * H, 16)]
            outbuf_v[i, pl.ds(0, 16)] = m

        pltpu.sync_copy(outbuf_v, acc_sh.at[idx_d[buf]], add=True)

    def chunk_pair(g, carry):
        process(2 * g, 0)
        process(2 * g + 1, 1)
        return carry
    lax.fori_loop(0, NCH // 2, chunk_pair, 0)

    plsc.subcore_barrier()
    pltpu.sync_copy(acc_sh.at[pl.ds(sid * NPT, NPT)],
                    out_hbm.at[cid, pl.ds(sid * NPT, NPT)])


_sc_msg = pl.kernel(
    _sc_msg_body,
    out_type=jax.ShapeDtypeStruct((NC, NPAD, AW), jnp.float32),
    mesh=plsc.VectorSubcoreMesh(core_axis_name="c", subcore_axis_name="s"),
    compiler_params=pltpu.CompilerParams(use_tc_tiling_on_sc=False),
    scratch_types=[
        pltpu.VMEM((CHUNK,), jnp.int32),
        pltpu.VMEM((CHUNK,), jnp.int32),
        pltpu.VMEM((CHUNK,), jnp.int32),
        pltpu.VMEM((CHUNK,), jnp.int32),
        pltpu.VMEM((CHUNK, H), jnp.float32),
        pltpu.VMEM((CHUNK, H), jnp.float32),
        pltpu.VMEM((CHUNK, PW), jnp.float32),
        pltpu.VMEM((CHUNK, PW), jnp.float32),
        pltpu.VMEM((CHUNK, AW), jnp.float32),
        pltpu.VMEM((CHUNK, AW), jnp.float32),
        pltpu.VMEM((NPT, AW), jnp.float32),
        pltpu.VMEM_SHARED((NPAD, AW), jnp.float32),
        pltpu.SemaphoreType.DMA,
        pltpu.SemaphoreType.DMA,
        pltpu.SemaphoreType.DMA,
        pltpu.SemaphoreType.DMA,
        pltpu.SemaphoreType.DMA,
        pltpu.SemaphoreType.DMA,
    ],
)


def _make_wcat(w2, b2, din):
    w2p = w2.reshape(H, din, H).transpose(1, 0, 2).reshape(din, HH)
    b2r = b2.reshape(din, H)
    return jnp.concatenate([w2p, b2r], axis=1)


@jax.jit
def kernel(x, edge_attr, edge_index, en1_w1, en1_b1, en1_w2, en1_b2,
           en2_w1, en2_b1, en2_w2, en2_b2, root1, bias1, root2, bias2,
           q_w, q_b):
    src = edge_index[0].reshape(NW, NCH, CHUNK)
    dst = edge_index[1].reshape(NW, NCH, CHUNK)

    wcat1 = _make_wcat(en1_w2, en1_b2, DIN)
    wcat2 = _make_wcat(en2_w2, en2_b2, H)

    eb = 16000
    xb = 1000
    z1, z2, p1 = pl.pallas_call(
        _pre_body,
        grid=(E // eb,),
        in_specs=[
            pl.BlockSpec((eb, DE), lambda i: (i, 0)),
            pl.BlockSpec((DE, H), lambda i: (0, 0)),
            pl.BlockSpec((1, H), lambda i: (0, 0)),
            pl.BlockSpec((DE, H), lambda i: (0, 0)),
            pl.BlockSpec((1, H), lambda i: (0, 0)),
            pl.BlockSpec((xb, DIN), lambda i: (i, 0)),
            pl.BlockSpec((DIN, PW), lambda i: (0, 0)),
        ],
        out_specs=[
            pl.BlockSpec((eb, H), lambda i: (i, 0)),
            pl.BlockSpec((eb, H), lambda i: (i, 0)),
            pl.BlockSpec((xb, PW), lambda i: (i, 0)),
        ],
        out_shape=[
            jax.ShapeDtypeStruct((E, H), jnp.float32),
            jax.ShapeDtypeStruct((E, H), jnp.float32),
            jax.ShapeDtypeStruct((N, PW), jnp.float32),
        ],
    )(edge_attr, en1_w1, en1_b1.reshape(1, H), en2_w1, en2_b1.reshape(1, H),
      x, wcat1)

    z1g = z1.reshape(NW, NCH, CHUNK, H)
    z2g = z2.reshape(NW, NCH, CHUNK, H)
    nb = 2000

    acc1 = _sc_msg(p1, z1g, src, dst)

    h, p2 = pl.pallas_call(
        _mid_body,
        grid=(N // nb,),
        in_specs=[
            pl.BlockSpec((NC, nb, AW), lambda i: (0, i, 0)),
            pl.BlockSpec((nb, DIN), lambda i: (i, 0)),
            pl.BlockSpec((DIN, H), lambda i: (0, 0)),
            pl.BlockSpec((1, H), lambda i: (0, 0)),
            pl.BlockSpec((H, PW), lambda i: (0, 0)),
        ],
        out_specs=[
            pl.BlockSpec((nb, H), lambda i: (i, 0)),
            pl.BlockSpec((nb, PW), lambda i: (i, 0)),
        ],
        out_shape=[
            jax.ShapeDtypeStruct((N, H), jnp.float32),
            jax.ShapeDtypeStruct((N, PW), jnp.float32),
        ],
    )(acc1, x, root1, bias1.reshape(1, H), wcat2)

    acc2 = _sc_msg(p2, z2g, src, dst)

    out2d = pl.pallas_call(
        _fin_body,
        grid=(N // nb,),
        in_specs=[
            pl.BlockSpec((NC, nb, AW), lambda i: (0, i, 0)),
            pl.BlockSpec((nb, H), lambda i: (i, 0)),
            pl.BlockSpec((H, H), lambda i: (0, 0)),
            pl.BlockSpec((1, H), lambda i: (0, 0)),
            pl.BlockSpec((H, 1), lambda i: (0, 0)),
            pl.BlockSpec((1, 1), lambda i: (0, 0)),
        ],
        out_specs=pl.BlockSpec((nb, 1), lambda i: (i, 0)),
        out_shape=jax.ShapeDtypeStruct((N, 1), jnp.float32),
    )(acc2, h, root2, bias2.reshape(1, H), q_w, q_b.reshape(1, 1))

    return out2d[:, 0]


# speedup vs baseline: 5.9775x; 1.0019x over previous
import functools

import jax
import jax.numpy as jnp
from jax import lax
from jax.experimental import pallas as pl
from jax.experimental.pallas import tpu as pltpu
from jax.experimental.pallas import tpu_sc as plsc

N = 10000
E = 160000
DIN = 128
DE = 16
H = 16
HH = H * H
PW = HH + H
GB = 9
PWB = 32 * GB
AW = 32
NPAD = 10240

NC = 2
NS = 16
NW = NC * NS
EPT = E // NW
CHUNK = 125
NCH = EPT // CHUNK
NPT = NPAD // NS


def _pre_body(ea_ref, w1a_ref, b1a_ref, w1b_ref, b1b_ref, x_ref, wcat_ref,
              z1_ref, z2_ref, p_ref):
    ea = ea_ref[...]
    z1_ref[...] = jnp.maximum(
        jnp.dot(ea, w1a_ref[...], preferred_element_type=jnp.float32)
        + b1a_ref[...], 0.0)
    z2_ref[...] = jnp.maximum(
        jnp.dot(ea, w1b_ref[...], preferred_element_type=jnp.float32)
        + b1b_ref[...], 0.0)
    p_ref[...] = jnp.dot(x_ref[...], wcat_ref[...],
                         preferred_element_type=jnp.float32)


def _mid_body(acc_ref, x_ref, root_ref, bias_ref, wcat_ref, h_ref, p_ref):
    s = acc_ref[0] + acc_ref[1]
    agg = s[:, 0:H] / jnp.maximum(s[:, H:H + 1], 1.0)
    h = jnp.maximum(
        agg + jnp.dot(x_ref[...], root_ref[...],
                      preferred_element_type=jnp.float32) + bias_ref[...], 0.0)
    h_ref[...] = h
    p_ref[...] = jnp.dot(h, wcat_ref[...], preferred_element_type=jnp.float32)


def _fin_body(acc_ref, h_ref, root_ref, bias_ref, qw_ref, qb_ref, out_ref):
    s = acc_ref[0] + acc_ref[1]
    agg = s[:, 0:H] / jnp.maximum(s[:, H:H + 1], 1.0)
    h2 = jnp.maximum(
        agg + jnp.dot(h_ref[...], root_ref[...],
                      preferred_element_type=jnp.float32) + bias_ref[...], 0.0)
    out_ref[...] = jnp.dot(h2, qw_ref[...],
                           preferred_element_type=jnp.float32) + qb_ref[0, 0]


def _sc_msg_body(p_hbm, z_hbm, src_hbm, dst_hbm, out_hbm,
                 idx_s0, idx_s1, idx_d0, idx_d1, z_v0, z_v1,
                 rows_v0, rows_v1, outbuf_v0, outbuf_v1, zero_v, acc_sh,
                 sem_r0, sem_r1, sem_z0, sem_z1, sem_d0, sem_d1):
    cid = lax.axis_index("c")
    sid = lax.axis_index("s")
    wid = cid * NS + sid

    idx_s = (idx_s0, idx_s1)
    idx_d = (idx_d0, idx_d1)
    z_bufs = (z_v0, z_v1)
    row_bufs = (rows_v0, rows_v1)
    out_bufs = (outbuf_v0, outbuf_v1)
    sem_r = (sem_r0, sem_r1)
    sem_z = (sem_z0, sem_z1)
    sem_d = (sem_d0, sem_d1)

    zvec = jnp.zeros((16,), jnp.float32)

    @plsc.parallel_loop(0, NPT, unroll=8)
    def _(i):
        zero_v[i, pl.ds(0, 16)] = zvec
        zero_v[i, pl.ds(16, 16)] = zvec
    pltpu.sync_copy(zero_v, acc_sh.at[pl.ds(sid * NPT, NPT)])

    pat = jnp.where(lax.iota(jnp.int32, 16) == 0, 1.0, 0.0).astype(jnp.float32)

    @plsc.parallel_loop(0, CHUNK, unroll=5)
    def _(i):
        outbuf_v0[i, pl.ds(H, 16)] = pat
        outbuf_v1[i, pl.ds(H, 16)] = pat

    plsc.subcore_barrier()

    def issue(c, buf):
        pltpu.sync_copy(src_hbm.at[wid, c], idx_s[buf])
        pltpu.async_copy(p_hbm.at[idx_s[buf]], row_bufs[buf], sem_r[buf])
        pltpu.async_copy(z_hbm.at[wid, c], z_bufs[buf], sem_z[buf])
        pltpu.async_copy(dst_hbm.at[wid, c], idx_d[buf], sem_d[buf])

    issue(0, 0)

    def process(c, buf):
        rows_v = row_bufs[buf]
        z_v = z_bufs[buf]
        outbuf_v = out_bufs[buf]

        @pl.when(c + 1 < NCH)
        def _():
            issue(c + 1, 1 - buf)

        pltpu.make_async_copy(p_hbm.at[idx_s[buf]], rows_v,
                              sem_r[buf]).wait()
        pltpu.make_async_copy(z_hbm.at[wid, c], z_v, sem_z[buf]).wait()
        pltpu.make_async_copy(dst_hbm.at[wid, c], idx_d[buf],
                              sem_d[buf]).wait()

        @plsc.parallel_loop(0, CHUNK, unroll=5)
        def _(i):
            zrow = z_v[i, pl.ds(0, H)]
            m = rows_v[i, pl.ds(HH, 16)]
            for k in range(H):
                m = m + zrow[k] * rows_v[i, pl.ds(k * H, 16)]
            outbuf_v[i, pl.ds(0, 16)] = m

        pltpu.sync_copy(outbuf_v, acc_sh.at[idx_d[buf]], add=True)

    def chunk_pair(g, carry):
        process(2 * g, 0)
        process(2 * g + 1, 1)
        return carry
    lax.fori_loop(0, NCH // 2, chunk_pair, 0)

    plsc.subcore_barrier()
    pltpu.sync_copy(acc_sh.at[pl.ds(sid * NPT, NPT)],
                    out_hbm.at[cid, pl.ds(sid * NPT, NPT)])


_sc_msg = pl.kernel(
    _sc_msg_body,
    out_type=jax.ShapeDtypeStruct((NC, NPAD, AW), jnp.float32),
    mesh=plsc.VectorSubcoreMesh(core_axis_name="c", subcore_axis_name="s"),
    compiler_params=pltpu.CompilerParams(use_tc_tiling_on_sc=False),
    scratch_types=[
        pltpu.VMEM((CHUNK,), jnp.int32),
        pltpu.VMEM((CHUNK,), jnp.int32),
        pltpu.VMEM((CHUNK,), jnp.int32),
        pltpu.VMEM((CHUNK,), jnp.int32),
        pltpu.VMEM((CHUNK, H), jnp.float32),
        pltpu.VMEM((CHUNK, H), jnp.float32),
        pltpu.VMEM((CHUNK, PW), jnp.float32),
        pltpu.VMEM((CHUNK, PW), jnp.float32),
        pltpu.VMEM((CHUNK, AW), jnp.float32),
        pltpu.VMEM((CHUNK, AW), jnp.float32),
        pltpu.VMEM((NPT, AW), jnp.float32),
        pltpu.VMEM_SHARED((NPAD, AW), jnp.float32),
        pltpu.SemaphoreType.DMA,
        pltpu.SemaphoreType.DMA,
        pltpu.SemaphoreType.DMA,
        pltpu.SemaphoreType.DMA,
        pltpu.SemaphoreType.DMA,
        pltpu.SemaphoreType.DMA,
    ],
)


def _make_wcat(w2, b2, din):
    w2p = w2.reshape(H, din, H).transpose(1, 0, 2).reshape(din, HH)
    b2r = b2.reshape(din, H)
    return jnp.concatenate([w2p, b2r], axis=1)


@jax.jit
def kernel(x, edge_attr, edge_index, en1_w1, en1_b1, en1_w2, en1_b2,
           en2_w1, en2_b1, en2_w2, en2_b2, root1, bias1, root2, bias2,
           q_w, q_b):
    src = edge_index[0].reshape(NW, NCH, CHUNK)
    dst = edge_index[1].reshape(NW, NCH, CHUNK)

    wcat1 = _make_wcat(en1_w2, en1_b2, DIN)
    wcat2 = _make_wcat(en2_w2, en2_b2, H)

    eb = 16000
    xb = 1000
    z1, z2, p1 = pl.pallas_call(
        _pre_body,
        grid=(E // eb,),
        in_specs=[
            pl.BlockSpec((eb, DE), lambda i: (i, 0)),
            pl.BlockSpec((DE, H), lambda i: (0, 0)),
            pl.BlockSpec((1, H), lambda i: (0, 0)),
            pl.BlockSpec((DE, H), lambda i: (0, 0)),
            pl.BlockSpec((1, H), lambda i: (0, 0)),
            pl.BlockSpec((xb, DIN), lambda i: (i, 0)),
            pl.BlockSpec((DIN, PW), lambda i: (0, 0)),
        ],
        out_specs=[
            pl.BlockSpec((eb, H), lambda i: (i, 0)),
            pl.BlockSpec((eb, H), lambda i: (i, 0)),
            pl.BlockSpec((xb, PW), lambda i: (i, 0)),
        ],
        out_shape=[
            jax.ShapeDtypeStruct((E, H), jnp.float32),
            jax.ShapeDtypeStruct((E, H), jnp.float32),
            jax.ShapeDtypeStruct((N, PW), jnp.float32),
        ],
    )(edge_attr, en1_w1, en1_b1.reshape(1, H), en2_w1, en2_b1.reshape(1, H),
      x, wcat1)

    z1g = z1.reshape(NW, NCH, CHUNK, H)
    z2g = z2.reshape(NW, NCH, CHUNK, H)
    nb = 2000

    acc1 = _sc_msg(p1, z1g, src, dst)

    h, p2 = pl.pallas_call(
        _mid_body,
        grid=(N // nb,),
        in_specs=[
            pl.BlockSpec((NC, nb, AW), lambda i: (0, i, 0)),
            pl.BlockSpec((nb, DIN), lambda i: (i, 0)),
            pl.BlockSpec((DIN, H), lambda i: (0, 0)),
            pl.BlockSpec((1, H), lambda i: (0, 0)),
            pl.BlockSpec((H, PW), lambda i: (0, 0)),
        ],
        out_specs=[
            pl.BlockSpec((nb, H), lambda i: (i, 0)),
            pl.BlockSpec((nb, PW), lambda i: (i, 0)),
        ],
        out_shape=[
            jax.ShapeDtypeStruct((N, H), jnp.float32),
            jax.ShapeDtypeStruct((N, PW), jnp.float32),
        ],
    )(acc1, x, root1, bias1.reshape(1, H), wcat2)

    acc2 = _sc_msg(p2, z2g, src, dst)

    out2d = pl.pallas_call(
        _fin_body,
        grid=(N // nb,),
        in_specs=[
            pl.BlockSpec((NC, nb, AW), lambda i: (0, i, 0)),
            pl.BlockSpec((nb, H), lambda i: (i, 0)),
            pl.BlockSpec((H, H), lambda i: (0, 0)),
            pl.BlockSpec((1, H), lambda i: (0, 0)),
            pl.BlockSpec((H, 1), lambda i: (0, 0)),
            pl.BlockSpec((1, 1), lambda i: (0, 0)),
        ],
        out_specs=pl.BlockSpec((nb, 1), lambda i: (i, 0)),
        out_shape=jax.ShapeDtypeStruct((N, 1), jnp.float32),
    )(acc2, h, root2, bias2.reshape(1, H), q_w, q_b.reshape(1, 1))

    return out2d[:, 0]
